# Initial kernel scaffold; baseline (speedup 1.0000x reference)
import jax
import jax.numpy as jnp
from jax.experimental import pallas as pl


def kernel(pred_scores, pred_boxes, pred_objectness, anchor_points, gt_boxes, gt_labels, valid_class_mask):
    raise NotImplementedError("write your pallas kernel here")


# speedup vs baseline: 116.0688x; 116.0688x over previous
import functools

import jax
import jax.numpy as jnp
from jax import lax
from jax.experimental import pallas as pl
from jax.experimental.pallas import tpu as pltpu

_CANDIDATE_TOPK = 8
_CENTER_RADIUS = 0.75
_DUP_RADIUS = 1.25
_VFL_ALPHA = 0.75
_VFL_GAMMA = 2.0
_BOX_WEIGHT = 2.5
_BN = 2000
_NEG_INF = float("-inf")
_BIG_I = 1 << 30


def _sig(x):
    return 1.0 / (1.0 + jnp.exp(-x))


def _gt_geom(gt_ref, ax, ay):
    x1 = gt_ref[0:1, :]
    y1 = gt_ref[1:2, :]
    x2 = gt_ref[2:3, :]
    y2 = gt_ref[3:4, :]
    inside = (ax >= x1) & (ax <= x2) & (ay >= y1) & (ay <= y2)
    cx = (x1 + x2) * 0.5
    cy = (y1 + y2) * 0.5
    hx = jnp.maximum((x2 - x1) * 0.5, 1.0)
    hy = jnp.maximum((y2 - y1) * 0.5, 1.0)
    dx = jnp.abs(ax - cx) / hx
    dy = jnp.abs(ay - cy) / hy
    dmax = jnp.maximum(dx, dy)
    d2 = dx * dx + dy * dy
    return inside, dmax, d2, (x1, y1, x2, y2)


def _anyc_body(ap_ref, gt_ref, out_ref, acc_ref):
    i = pl.program_id(0)
    ax = ap_ref[:, 0:1]
    ay = ap_ref[:, 1:2]
    inside, dmax, _, _ = _gt_geom(gt_ref, ax, ay)
    cc = inside & (dmax <= _CENTER_RADIUS)
    part = jnp.max(jnp.where(cc, 1.0, 0.0), axis=0, keepdims=True)

    @pl.when(i == 0)
    def _init():
        acc_ref[...] = jnp.zeros_like(acc_ref)

    acc_ref[...] = jnp.maximum(acc_ref[...], part)

    @pl.when(i == pl.num_programs(0) - 1)
    def _fin():
        out_ref[...] = acc_ref[...]


def _topk_body(scores_ref, boxes_ref, obj_ref, ap_ref, gt_ref, lab_ref,
               vcm_ref, anyc_ref, vals_out, idx_out, iou_out,
               sv_ref, si_ref, so_ref, *, nc, num_classes):
    i = pl.program_id(0)
    bn = scores_ref.shape[0]
    g = lab_ref.shape[1]

    ax = ap_ref[:, 0:1]
    ay = ap_ref[:, 1:2]
    inside, dmax, d2, (x1, y1, x2, y2) = _gt_geom(gt_ref, ax, ay)
    ccf = jnp.where(inside & (dmax <= _CENTER_RADIUS), 1.0, 0.0)
    insidef = jnp.where(inside, 1.0, 0.0)
    anycf = anyc_ref[...]
    candf = anycf * ccf + (1.0 - anycf) * insidef
    prior = jnp.exp(-0.5 * d2)

    px1 = boxes_ref[:, 0:1]
    py1 = boxes_ref[:, 1:2]
    px2 = boxes_ref[:, 2:3]
    py2 = boxes_ref[:, 3:4]
    ix1 = jnp.maximum(px1, x1)
    iy1 = jnp.maximum(py1, y1)
    ix2 = jnp.minimum(px2, x2)
    iy2 = jnp.minimum(py2, y2)
    inter = jnp.maximum(ix2 - ix1, 0.0) * jnp.maximum(iy2 - iy1, 0.0)
    area_a = jnp.maximum(px2 - px1, 0.0) * jnp.maximum(py2 - py1, 0.0)
    area_b = jnp.maximum(x2 - x1, 0.0) * jnp.maximum(y2 - y1, 0.0)
    iou = inter / (area_a + area_b - inter + 1e-7)

    lab = lab_ref[...]
    onehot = jnp.where(
        lax.broadcasted_iota(jnp.int32, (num_classes, g), 0) == lab, 1.0, 0.0)
    logit = jnp.dot(scores_ref[...], onehot, preferred_element_type=jnp.float32)
    cls_s = _sig(logit)
    po = _sig(obj_ref[...])
    vcm_g = jnp.dot(vcm_ref[...], onehot, preferred_element_type=jnp.float32)
    vgtf = jnp.where((lab >= 0) & (lab < num_classes), 1.0, 0.0) * jnp.where(
        vcm_g > 0.0, 1.0, 0.0)

    quality = jnp.sqrt(jnp.maximum(po * cls_s, 0.0))
    metric = quality * (iou * iou) * (prior * prior)
    m = jnp.where(candf * vgtf > 0.0, metric, _NEG_INF)
    gidx = i * bn + lax.broadcasted_iota(jnp.int32, (bn, g), 0)

    for k in range(_CANDIDATE_TOPK):
        mx = jnp.max(m, axis=0, keepdims=True)
        pick = m == mx
        cidx = jnp.min(jnp.where(pick, gidx, _BIG_I), axis=0, keepdims=True)
        sel = pick & (gidx == cidx)
        sv_ref[8 + k:9 + k, :] = mx
        si_ref[8 + k:9 + k, :] = cidx
        so_ref[8 + k:9 + k, :] = jnp.sum(
            jnp.where(sel, iou, 0.0), axis=0, keepdims=True)
        m = jnp.where(sel, _NEG_INF, m)

    @pl.when(i == 0)
    def _init():
        sv_ref[0:8, :] = jnp.full((8, g), _NEG_INF, jnp.float32)
        si_ref[0:8, :] = jnp.full((8, g), _BIG_I, jnp.int32)
        so_ref[0:8, :] = jnp.zeros((8, g), jnp.float32)

    v = sv_ref[...]
    ix = si_ref[...]
    io = so_ref[...]
    outv, outi, outo = [], [], []
    for k in range(_CANDIDATE_TOPK):
        mx = jnp.max(v, axis=0, keepdims=True)
        pick = v == mx
        cidx = jnp.min(jnp.where(pick, ix, _BIG_I), axis=0, keepdims=True)
        sel = pick & (ix == cidx)
        outv.append(mx)
        outi.append(cidx)
        outo.append(jnp.sum(jnp.where(sel, io, 0.0), axis=0, keepdims=True))
        v = jnp.where(sel, _NEG_INF, v)
    tv = jnp.concatenate(outv, axis=0)
    ti = jnp.concatenate(outi, axis=0)
    to = jnp.concatenate(outo, axis=0)
    sv_ref[0:8, :] = tv
    si_ref[0:8, :] = ti
    so_ref[0:8, :] = to

    @pl.when(i == nc - 1)
    def _fin():
        vals_out[...] = tv
        idx_out[...] = ti
        iou_out[...] = to


def _greedy_body(vals_ref, idx_ref, iou_ref, lab_ref,
                 mp_ref, mlab_ref, movl_ref, mval_ref, gmask_ref, scal_ref):
    g = lab_ref.shape[1]
    idx_arr = idx_ref[...]
    iou_arr = iou_ref[...]
    labels = lab_ref[...]
    colg = lax.broadcasted_iota(jnp.int32, (8, g), 1)
    rowk = lax.broadcasted_iota(jnp.int32, (8, g), 0)
    fkey = colg * 8 + rowk
    gcol = lax.broadcasted_iota(jnp.int32, (1, g), 1)
    iota128 = lax.broadcasted_iota(jnp.int32, (1, 128), 1)

    def body(t, st):
        v, pv, lv, ov, tv, gm, bsum, cnt = st
        mx = jnp.max(v)
        take = mx > _NEG_INF
        fsel = jnp.min(jnp.where(v == mx, fkey, _BIG_I))
        picked = (v == mx) & (fkey == fsel)
        p0 = jnp.sum(jnp.where(picked, idx_arr, 0))
        g0 = fsel // 8
        i0 = jnp.sum(jnp.where(picked, iou_arr, 0.0))
        l0 = jnp.sum(jnp.where(gcol == g0, labels, 0))
        v = jnp.where(take & ((colg == g0) | (idx_arr == p0)), _NEG_INF, v)
        slot = (iota128 == t) & take
        pv = jnp.where(slot, p0, pv)
        lv = jnp.where(slot, l0, lv)
        ov = jnp.where(slot, i0, ov)
        tv = jnp.where(slot, 1.0, tv)
        gm = jnp.where(take & (gcol == g0), 1.0, gm)
        bsum = bsum + jnp.where(take, 1.0 - i0, 0.0)
        cnt = cnt + jnp.where(take, 1.0, 0.0)
        return (v, pv, lv, ov, tv, gm, bsum, cnt)

    init = (vals_ref[...],
            jnp.full((1, 128), -1, jnp.int32),
            jnp.zeros((1, 128), jnp.int32),
            jnp.zeros((1, 128), jnp.float32),
            jnp.zeros((1, 128), jnp.float32),
            jnp.zeros((1, g), jnp.float32),
            jnp.float32(0.0),
            jnp.float32(0.0))
    _, pv, lv, ov, tv, gm, bsum, cnt = lax.fori_loop(0, g, body, init)
    mp_ref[...] = pv
    mlab_ref[...] = lv
    movl_ref[...] = ov
    mval_ref[...] = tv
    gmask_ref[...] = gm
    scal_ref[...] = (bsum * jnp.where(iota128 == 0, 1.0, 0.0)
                     + cnt * jnp.where(iota128 == 1, 1.0, 0.0))


def _loss_body(scores_ref, obj_ref, ap_ref, gt_ref, mp_ref, mlab_ref,
               movl_ref, mval_ref, gmask_ref, out_ref, acc_ref, *, nc):
    i = pl.program_id(0)
    bn = scores_ref.shape[0]
    num_classes = scores_ref.shape[1]

    niota = mp_ref[...] * 0 + i * bn + lax.broadcasted_iota(
        jnp.int32, (bn, 128), 0)
    eq = (niota == mp_ref[...]) & (mval_ref[...] > 0.0)
    eqf = jnp.where(eq, 1.0, 0.0)
    fgf = jnp.max(eqf, axis=1, keepdims=True)
    mlabel = jnp.sum(jnp.where(eq, mlab_ref[...], 0), axis=1, keepdims=True)
    movl = jnp.sum(eqf * movl_ref[...], axis=1, keepdims=True)
    tval = jnp.maximum(movl, 0.1) * fgf

    x = scores_ref[...]
    eqc = lax.broadcasted_iota(jnp.int32, (bn, num_classes), 1) == mlabel
    t = jnp.where(eqc, tval, 0.0) * fgf
    prob = jax.nn.sigmoid(x)
    w = _VFL_ALPHA * prob * prob * (1.0 - t) + t
    bce = jnp.maximum(x, 0.0) - x * t + jnp.log1p(jnp.exp(-jnp.abs(x)))
    cls_sum = jnp.sum(bce * w)

    ax = ap_ref[:, 0:1]
    ay = ap_ref[:, 1:2]
    inside, dmax, _, _ = _gt_geom(gt_ref, ax, ay)
    dc = inside & (dmax <= _DUP_RADIUS) & (gmask_ref[...] > 0.0)
    dupany = jnp.max(jnp.where(dc, 1.0, 0.0), axis=1, keepdims=True)
    dup = dupany * (1.0 - fgf)

    ox = obj_ref[...]
    bobj = (jnp.maximum(ox, 0.0) - ox * fgf
            + jnp.log1p(jnp.exp(-jnp.abs(ox))))
    obj_sum = jnp.sum(bobj * (1.0 - dup))

    iota128 = lax.broadcasted_iota(jnp.int32, (1, 128), 1)
    part = (cls_sum * jnp.where(iota128 == 0, 1.0, 0.0)
            + obj_sum * jnp.where(iota128 == 1, 1.0, 0.0))

    @pl.when(i == 0)
    def _init():
        acc_ref[...] = jnp.zeros_like(acc_ref)

    acc_ref[...] = acc_ref[...] + part

    @pl.when(i == nc - 1)
    def _fin():
        out_ref[...] = acc_ref[...]


def kernel(pred_scores, pred_boxes, pred_objectness, anchor_points,
           gt_boxes, gt_labels, valid_class_mask):
    n, num_classes = pred_scores.shape
    g = gt_boxes.shape[0]
    nc = n // _BN
    obj2d = pred_objectness.reshape(n, 1)
    gt_t = gt_boxes.T
    lab2d = gt_labels.reshape(1, g)
    vcm2d = valid_class_mask.astype(jnp.float32).reshape(1, num_classes)

    anyc = pl.pallas_call(
        _anyc_body,
        grid=(nc,),
        in_specs=[
            pl.BlockSpec((_BN, 2), lambda i: (i, 0)),
            pl.BlockSpec((4, g), lambda i: (0, 0)),
        ],
        out_specs=pl.BlockSpec((1, g), lambda i: (0, 0)),
        out_shape=jax.ShapeDtypeStruct((1, g), jnp.float32),
        scratch_shapes=[pltpu.VMEM((1, g), jnp.float32)],
    )(anchor_points, gt_t)

    vals8, idx8, iou8 = pl.pallas_call(
        functools.partial(_topk_body, nc=nc, num_classes=num_classes),
        grid=(nc,),
        in_specs=[
            pl.BlockSpec((_BN, num_classes), lambda i: (i, 0)),
            pl.BlockSpec((_BN, 4), lambda i: (i, 0)),
            pl.BlockSpec((_BN, 1), lambda i: (i, 0)),
            pl.BlockSpec((_BN, 2), lambda i: (i, 0)),
            pl.BlockSpec((4, g), lambda i: (0, 0)),
            pl.BlockSpec((1, g), lambda i: (0, 0)),
            pl.BlockSpec((1, num_classes), lambda i: (0, 0)),
            pl.BlockSpec((1, g), lambda i: (0, 0)),
        ],
        out_specs=[
            pl.BlockSpec((8, g), lambda i: (0, 0)),
            pl.BlockSpec((8, g), lambda i: (0, 0)),
            pl.BlockSpec((8, g), lambda i: (0, 0)),
        ],
        out_shape=[
            jax.ShapeDtypeStruct((8, g), jnp.float32),
            jax.ShapeDtypeStruct((8, g), jnp.int32),
            jax.ShapeDtypeStruct((8, g), jnp.float32),
        ],
        scratch_shapes=[
            pltpu.VMEM((16, g), jnp.float32),
            pltpu.VMEM((16, g), jnp.int32),
            pltpu.VMEM((16, g), jnp.float32),
        ],
    )(pred_scores, pred_boxes, obj2d, anchor_points, gt_t, lab2d, vcm2d, anyc)

    mp, mlab, movl, mval, gmask, scal = pl.pallas_call(
        _greedy_body,
        out_shape=[
            jax.ShapeDtypeStruct((1, 128), jnp.int32),
            jax.ShapeDtypeStruct((1, 128), jnp.int32),
            jax.ShapeDtypeStruct((1, 128), jnp.float32),
            jax.ShapeDtypeStruct((1, 128), jnp.float32),
            jax.ShapeDtypeStruct((1, g), jnp.float32),
            jax.ShapeDtypeStruct((1, 128), jnp.float32),
        ],
    )(vals8, idx8, iou8, lab2d)

    sums = pl.pallas_call(
        functools.partial(_loss_body, nc=nc),
        grid=(nc,),
        in_specs=[
            pl.BlockSpec((_BN, num_classes), lambda i: (i, 0)),
            pl.BlockSpec((_BN, 1), lambda i: (i, 0)),
            pl.BlockSpec((_BN, 2), lambda i: (i, 0)),
            pl.BlockSpec((4, g), lambda i: (0, 0)),
            pl.BlockSpec((1, 128), lambda i: (0, 0)),
            pl.BlockSpec((1, 128), lambda i: (0, 0)),
            pl.BlockSpec((1, 128), lambda i: (0, 0)),
            pl.BlockSpec((1, 128), lambda i: (0, 0)),
            pl.BlockSpec((1, g), lambda i: (0, 0)),
        ],
        out_specs=pl.BlockSpec((1, 128), lambda i: (0, 0)),
        out_shape=jax.ShapeDtypeStruct((1, 128), jnp.float32),
        scratch_shapes=[pltpu.VMEM((1, 128), jnp.float32)],
    )(pred_scores, obj2d, anchor_points, gt_t, mp, mlab, movl, mval, gmask)

    box_sum = scal[0, 0]
    cnt = scal[0, 1]
    num_fg = jnp.maximum(cnt, 1.0)
    return (sums[0, 0] + _BOX_WEIGHT * box_sum + sums[0, 1]) / num_fg
